# Initial kernel scaffold; baseline (speedup 1.0000x reference)
import jax
import jax.numpy as jnp
from jax.experimental import pallas as pl


def kernel(x, edge_index, edge_attr, batch, GM_input, params):
    raise NotImplementedError("write your pallas kernel here")


# speedup vs baseline: 5.2014x; 5.2014x over previous
import functools

import jax
import jax.numpy as jnp
from jax import lax
from jax.experimental import pallas as pl
from jax.experimental.pallas import tpu as pltpu
from jax.experimental.pallas import tpu_sc as plsc

N = 10000
NPAD = 10240
E = 320000
E2 = E + N
CH = 128
NW = 32
EPB = NW * CH
E0_PAD = ((E + EPB - 1) // EPB) * EPB
E2_PAD = ((E2 + EPB - 1) // EPB) * EPB
B = 64
T = 100
DE = 16
F32 = jnp.float32


def _sc_mesh():
    return plsc.VectorSubcoreMesh(core_axis_name="c", subcore_axis_name="s")


def _zero_rows(rows, width):
    def body(i, c):
        for kk in range(width // 16):
            rows[i, pl.ds(kk * 16, 16)] = jnp.zeros((16,), F32)
        return c
    lax.fori_loop(0, CH, body, 0)


def _make_deg_kernel():
    W = 32
    n_chunks = E0_PAD // EPB

    @functools.partial(
        pl.kernel,
        out_type=jax.ShapeDtypeStruct((2, NPAD, W), F32),
        mesh=_sc_mesh(),
        compiler_params=pltpu.CompilerParams(needs_layout_passes=False, use_tc_tiling_on_sc=False),
        scratch_types=[
            pltpu.VMEM((CH,), jnp.int32),
            pltpu.VMEM((CH, DE), F32),
            pltpu.VMEM((CH, W), F32),
            pltpu.VMEM_SHARED((NPAD, W), F32),
        ],
    )
    def k(d_hbm, attr_hbm, out_hbm, didx, attr, rows, acc):
        core = lax.axis_index("c")
        sub = lax.axis_index("s")
        _zero_rows(rows, W)
        rpt = NPAD // 16
        for j in range(rpt // CH):
            pltpu.sync_copy(rows, acc.at[pl.ds(sub * rpt + j * CH, CH)])
        plsc.subcore_barrier()
        lane = lax.iota(jnp.int32, 16)

        def chunk(ci, c):
            g = ci * NW + sub * 2 + core
            base = g * CH
            pltpu.sync_copy(d_hbm.at[pl.ds(base, CH)], didx)
            pltpu.sync_copy(attr_hbm.at[pl.ds(base, CH)], attr)

            def edge(i, cc):
                m = jnp.where(base + i < E, jnp.float32(1.0), jnp.float32(0.0))
                rows[i, pl.ds(0, 16)] = m * attr[i, pl.ds(0, 16)]
                rows[i, pl.ds(16, 16)] = jnp.where(lane == 0, m, jnp.float32(0.0))
                return cc
            lax.fori_loop(0, CH, edge, 0)
            pltpu.sync_copy(rows, acc.at[didx], add=True)
            return c
        lax.fori_loop(0, n_chunks, chunk, 0)

        plsc.subcore_barrier()
        for j in range(rpt // CH):
            sl = pl.ds(sub * rpt + j * CH, CH)
            pltpu.sync_copy(acc.at[sl], out_hbm.at[core, sl])

    return k


def _make_gat_sc_kernel(dout):
    W = dout + 16
    n_chunks = E2_PAD // EPB
    nk = dout // 16

    @functools.partial(
        pl.kernel,
        out_type=jax.ShapeDtypeStruct((2, NPAD, W), F32),
        mesh=_sc_mesh(),
        compiler_params=pltpu.CompilerParams(needs_layout_passes=False, use_tc_tiling_on_sc=False),
        scratch_types=[
            pltpu.VMEM((CH,), jnp.int32),
            pltpu.VMEM((CH,), jnp.int32),
            pltpu.VMEM((CH, dout), F32),
            pltpu.VMEM((CH, dout), F32),
            pltpu.VMEM((CH, dout), F32),
            pltpu.VMEM((CH, W), F32),
            pltpu.VMEM((dout, 16), F32),
            pltpu.VMEM_SHARED((NPAD, W), F32),
            pltpu.SemaphoreType.DMA,
            pltpu.SemaphoreType.DMA,
        ],
    )
    def k(xl_hbm, xr_hbm, xe_hbm, s_hbm, d_hbm, att_hbm, out_hbm,
          sidx, didx, xls, xrd, xec, rows, attv, acc, sem1, sem2):
        core = lax.axis_index("c")
        sub = lax.axis_index("s")
        _zero_rows(rows, W)
        rpt = NPAD // 16
        for j in range(rpt // CH):
            pltpu.sync_copy(rows, acc.at[pl.ds(sub * rpt + j * CH, CH)])
        pltpu.sync_copy(att_hbm, attv)
        plsc.subcore_barrier()
        lane = lax.iota(jnp.int32, 16)

        def chunk(ci, c):
            g = ci * NW + sub * 2 + core
            base = g * CH
            pltpu.sync_copy(s_hbm.at[pl.ds(base, CH)], sidx)
            pltpu.sync_copy(d_hbm.at[pl.ds(base, CH)], didx)
            cp1 = pltpu.async_copy(xl_hbm.at[sidx], xls, sem1)
            cp2 = pltpu.async_copy(xr_hbm.at[didx], xrd, sem2)
            pltpu.sync_copy(xe_hbm.at[pl.ds(base, CH)], xec)
            cp1.wait()
            cp2.wait()

            def group(gi, cc):
                rowi = gi * 16 + lane
                svec = jnp.zeros((16,), F32)
                for k in range(dout):
                    kf = jnp.full((16,), k, jnp.int32)
                    a = (plsc.load_gather(xls, [rowi, kf])
                         + plsc.load_gather(xrd, [rowi, kf])
                         + plsc.load_gather(xec, [rowi, kf]))
                    z = jnp.where(a > 0, a, jnp.float32(0.2) * a)
                    svec = svec + z * attv[k, pl.ds(0, 16)]
                e = jnp.exp(svec)
                e = jnp.where(base + rowi < E2, e, jnp.float32(0.0))
                for k in range(dout):
                    kf = jnp.full((16,), k, jnp.int32)
                    plsc.store_scatter(rows, [rowi, kf],
                                       e * plsc.load_gather(xls, [rowi, kf]))
                plsc.store_scatter(rows, [rowi, jnp.full((16,), dout, jnp.int32)], e)
                return cc
            lax.fori_loop(0, CH // 16, group, 0)
            pltpu.sync_copy(rows, acc.at[didx], add=True)
            return c
        lax.fori_loop(0, n_chunks, chunk, 0)

        plsc.subcore_barrier()
        for j in range(rpt // CH):
            sl = pl.ds(sub * rpt + j * CH, CH)
            pltpu.sync_copy(acc.at[sl], out_hbm.at[core, sl])

    return k


def _dotT(a, w):
    return lax.dot_general(a, w, (((1,), (1,)), ((), ())),
                           preferred_element_type=F32)


def _node1_body(p_ref, x_ref, wl_ref, bl_ref, wr_ref, br_ref,
                xl_ref, xr_ref, la_ref):
    s = p_ref[0] + p_ref[1]
    cnt = s[:, 16:17]
    la_ref[...] = s[:, 0:16] / jnp.maximum(cnt, 1.0)
    x = x_ref[...]
    xl_ref[...] = _dotT(x, wl_ref[...]) + bl_ref[...]
    xr_ref[...] = _dotT(x, wr_ref[...]) + br_ref[...]


def _make_node1():
    return pl.pallas_call(
        _node1_body,
        out_shape=(
            jax.ShapeDtypeStruct((NPAD, 32), F32),
            jax.ShapeDtypeStruct((NPAD, 32), F32),
            jax.ShapeDtypeStruct((NPAD, 16), F32),
        ),
    )


def _make_node(din, dout):
    W = din + 16

    def body(p_ref, bprev_ref, wl_ref, bl_ref, wr_ref, br_ref,
             xl_ref, xr_ref, h_ref):
        s = p_ref[0] + p_ref[1]
        den = jnp.maximum(s[:, din:din + 1], 1e-30)
        h = jnp.maximum(s[:, 0:din] / den + bprev_ref[...], 0.0)
        h_ref[...] = h
        xl_ref[...] = _dotT(h, wl_ref[...]) + bl_ref[...]
        xr_ref[...] = _dotT(h, wr_ref[...]) + br_ref[...]

    return pl.pallas_call(
        body,
        out_shape=(
            jax.ShapeDtypeStruct((NPAD, dout), F32),
            jax.ShapeDtypeStruct((NPAD, dout), F32),
            jax.ShapeDtypeStruct((NPAD, din), F32),
        ),
    )


def _make_edge(dout):
    BLK = 4096

    def body(ea_ref, we_ref, xe_ref):
        xe_ref[...] = _dotT(ea_ref[...], we_ref[...])

    return pl.pallas_call(
        body,
        grid=(E2_PAD // BLK,),
        in_specs=[
            pl.BlockSpec((BLK, DE), lambda i: (i, 0)),
            pl.BlockSpec((dout, DE), lambda i: (0, 0)),
        ],
        out_specs=pl.BlockSpec((BLK, dout), lambda i: (i, 0)),
        out_shape=jax.ShapeDtypeStruct((E2_PAD, dout), F32),
    )


def _lstm_gates(g, c):
    i = jax.nn.sigmoid(g[:, 0:64])
    f = jax.nn.sigmoid(g[:, 64:128])
    gg = jnp.tanh(g[:, 128:192])
    o = jax.nn.sigmoid(g[:, 192:256])
    c = f * c + i * gg
    h = o * jnp.tanh(c)
    return h, c


def _head_body(p_ref, b3_ref, batch_ref, gmt_ref,
               wih0_ref, whh0_ref, bb0_ref,
               wih1_ref, whh1_ref, bb1_ref,
               wih2_ref, whh2_ref, bb2_ref,
               fc1w_ref, fc1b_ref, fc2w_ref, fc2b_ref,
               g2sv_ref, g2svb_ref, g2so_ref, g2sob_ref,
               s2gv_ref, s2gvb_ref, s2go_ref, s2gob_ref,
               fusw_ref, fusb_ref, o1w_ref, o1b_ref, o2w_ref, o2b_ref,
               muw_ref, mub_ref, lvw_ref, lvb_ref,
               mu_ref, var_ref, ys0_ref, ys1_ref):
    s = p_ref[0] + p_ref[1]
    den = jnp.maximum(s[:, 16:17], 1e-30)
    h3 = s[:, 0:16] / den + b3_ref[...]
    gid = batch_ref[...]
    oh = (gid == lax.broadcasted_iota(jnp.int32, (NPAD, B), 1)).astype(F32)
    sums = lax.dot_general(oh, h3, (((0,), (0,)), ((), ())),
                           preferred_element_type=F32)
    cnt = jnp.sum(oh, axis=0)[:, None]
    gnn_emb = sums / jnp.maximum(cnt, 1.0)

    zero = jnp.zeros((B, 64), F32)

    def step0(t, hc):
        h, c = hc
        x_t = gmt_ref[t]
        g = x_t * wih0_ref[...] + _dotT(h, whh0_ref[...]) + bb0_ref[...]
        h, c = _lstm_gates(g, c)
        ys0_ref[t] = h
        return (h, c)
    lax.fori_loop(0, T, step0, (zero, zero))

    def step1(t, hc):
        h, c = hc
        g = (_dotT(ys0_ref[t], wih1_ref[...]) + _dotT(h, whh1_ref[...])
             + bb1_ref[...])
        h, c = _lstm_gates(g, c)
        ys1_ref[t] = h
        return (h, c)
    lax.fori_loop(0, T, step1, (zero, zero))

    def step2(t, hc):
        h, c = hc
        g = (_dotT(ys1_ref[t], wih2_ref[...]) + _dotT(h, whh2_ref[...])
             + bb2_ref[...])
        return _lstm_gates(g, c)
    h2f, _ = lax.fori_loop(0, T, step2, (zero, zero))

    gm = jnp.maximum(_dotT(h2f, fc1w_ref[...]) + fc1b_ref[...], 0.0)
    gm_out = _dotT(gm, fc2w_ref[...]) + fc2b_ref[...]

    sa = _dotT(_dotT(gnn_emb, g2sv_ref[...]) + g2svb_ref[...],
               g2so_ref[...]) + g2sob_ref[...]
    ga = _dotT(_dotT(gm_out, s2gv_ref[...]) + s2gvb_ref[...],
               s2go_ref[...]) + s2gob_ref[...]
    fused = (_dotT(jnp.concatenate([sa, ga], axis=1), fusw_ref[...])
             + fusb_ref[...])
    xcat = jnp.concatenate([gm_out, fused], axis=1)
    hh = jnp.maximum(_dotT(xcat, o1w_ref[...]) + o1b_ref[...], 0.0)
    hh = jnp.maximum(_dotT(hh, o2w_ref[...]) + o2b_ref[...], 0.0)
    mu_ref[...] = (jnp.sum(hh * muw_ref[...], axis=1, keepdims=True)
                   + mub_ref[...])
    var_ref[...] = jnp.exp(jnp.sum(hh * lvw_ref[...], axis=1, keepdims=True)
                           + lvb_ref[...])


def _make_head():
    return pl.pallas_call(
        _head_body,
        out_shape=(
            jax.ShapeDtypeStruct((B, 1), F32),
            jax.ShapeDtypeStruct((B, 1), F32),
        ),
        scratch_shapes=[
            pltpu.VMEM((T, B, 64), F32),
            pltpu.VMEM((T, B, 64), F32),
        ],
    )


def _pad_rows(a, rows):
    return jnp.pad(a, ((0, rows - a.shape[0]),) + ((0, 0),) * (a.ndim - 1))


def _att16(att):
    return jnp.tile(att[:, None], (1, 16))


def kernel(x, edge_index, edge_attr, batch, GM_input, params):
    src, dst = edge_index[0], edge_index[1]
    ar = jnp.arange(N, dtype=src.dtype)
    s2 = _pad_rows(jnp.concatenate([src, ar]), E2_PAD)
    d2 = _pad_rows(jnp.concatenate([dst, ar]), E2_PAD)
    d0 = _pad_rows(dst, E0_PAD)
    attr0 = _pad_rows(edge_attr, E0_PAD)
    xp = _pad_rows(x, NPAD)
    batch2d = jnp.pad(batch[:, None], ((0, NPAD - N), (0, 0)),
                      constant_values=B)
    gmt = jnp.transpose(GM_input[:, 0, :], (1, 0))[:, :, None]

    deg = _make_deg_kernel()(d0, attr0)

    p1, p2, p3 = params["conv1"], params["conv2"], params["conv3"]
    xl1, xr1, loop_attr = _make_node1()(
        deg, xp, p1["Wl"], p1["bl"][None, :], p1["Wr"], p1["br"][None, :])
    ea = jnp.concatenate(
        [edge_attr, loop_attr[:N], jnp.zeros((E2_PAD - E2, DE), F32)], axis=0)

    xe1 = _make_edge(32)(ea, p1["We"])
    acc1 = _make_gat_sc_kernel(32)(xl1, xr1, xe1, s2, d2, _att16(p1["att"]))

    xl2, xr2, _h1 = _make_node(32, 64)(
        acc1, p1["bias"][None, :], p2["Wl"], p2["bl"][None, :],
        p2["Wr"], p2["br"][None, :])
    xe2 = _make_edge(64)(ea, p2["We"])
    acc2 = _make_gat_sc_kernel(64)(xl2, xr2, xe2, s2, d2, _att16(p2["att"]))

    xl3, xr3, _h2 = _make_node(64, 16)(
        acc2, p2["bias"][None, :], p3["Wl"], p3["bl"][None, :],
        p3["Wr"], p3["br"][None, :])
    xe3 = _make_edge(16)(ea, p3["We"])
    acc3 = _make_gat_sc_kernel(16)(xl3, xr3, xe3, s2, d2, _att16(p3["att"]))

    l0, l1, l2 = params["lstm0"], params["lstm1"], params["lstm2"]
    mu, var = _make_head()(
        acc3, p3["bias"][None, :], batch2d, gmt,
        l0["Wih"][:, 0][None, :], l0["Whh"],
        (l0["bih"] + l0["bhh"])[None, :],
        l1["Wih"], l1["Whh"], (l1["bih"] + l1["bhh"])[None, :],
        l2["Wih"], l2["Whh"], (l2["bih"] + l2["bhh"])[None, :],
        params["fc1"]["W"], params["fc1"]["b"][None, :],
        params["fc2"]["W"], params["fc2"]["b"][None, :],
        params["g2s"]["Wv"], params["g2s"]["bv"][None, :],
        params["g2s"]["Wo"], params["g2s"]["bo"][None, :],
        params["s2g"]["Wv"], params["s2g"]["bv"][None, :],
        params["s2g"]["Wo"], params["s2g"]["bo"][None, :],
        params["fus"]["W"], params["fus"]["b"][None, :],
        params["o1"]["W"], params["o1"]["b"][None, :],
        params["o2"]["W"], params["o2"]["b"][None, :],
        params["mu"]["W"], params["mu"]["b"][None, :],
        params["lv"]["W"], params["lv"]["b"][None, :],
    )
    ones = jnp.ones((B, 4, 1, 1), F32)
    return mu, var, ones, ones
